# baseline (speedup 1.0000x reference)
import functools

import jax
import jax.numpy as jnp
from jax import lax
from jax.experimental import pallas as pl
from jax.experimental.pallas import tpu as pltpu
from jax.experimental.pallas import tpu_sc as plsc

_N = 1024 * 1024
_INV_T = 10.0

_NC = 2
_NS = 16
_NW = _NC * _NS
_CHUNK = _N // _NW
_L = 16
_UNROLL = 8
_VECS = _CHUNK // _L


def _sc_body(w_hbm, z_hbm, out_hbm, w_v, z_v, o_v):
    wid = lax.axis_index("s") * _NC + lax.axis_index("c")
    base = wid * _CHUNK
    pltpu.sync_copy(w_hbm.at[pl.ds(base, _CHUNK)], w_v)
    pltpu.sync_copy(z_hbm.at[pl.ds(base, _CHUNK)], z_v)

    def step(j, carry):
        off0 = j * (_L * _UNROLL)
        for u in range(_UNROLL):
            off = off0 + u * _L
            x = (w_v[pl.ds(off, _L)] - z_v[pl.ds(off, _L)]) * _INV_T
            o_v[pl.ds(off, _L)] = 1.0 / (1.0 + jnp.exp(-x))
        return carry

    lax.fori_loop(0, _VECS // _UNROLL, step, 0)
    pltpu.sync_copy(o_v, out_hbm.at[pl.ds(base, _CHUNK)])


def kernel(weights, noises):
    mesh = plsc.VectorSubcoreMesh(core_axis_name="c", subcore_axis_name="s")
    run = pl.kernel(
        _sc_body,
        mesh=mesh,
        out_type=jax.ShapeDtypeStruct((_N,), jnp.float32),
        scratch_types=[
            pltpu.VMEM((_CHUNK,), jnp.float32),
            pltpu.VMEM((_CHUNK,), jnp.float32),
            pltpu.VMEM((_CHUNK,), jnp.float32),
        ],
    )
    return run(weights, noises)


# speedup vs baseline: 1.0099x; 1.0099x over previous
import functools

import jax
import jax.numpy as jnp
from jax import lax
from jax.experimental import pallas as pl
from jax.experimental.pallas import tpu as pltpu
from jax.experimental.pallas import tpu_sc as plsc

_N = 1024 * 1024
_INV_T = 10.0

_NC = 2
_NS = 16
_NW = _NC * _NS
_CHUNK = _N // _NW
_L = 16
_UNROLL = 8
_VECS = _CHUNK // _L


def _sc_body(w_hbm, z_hbm, out_hbm, w_v, z_v, o_v):
    wid = lax.axis_index("s") * _NC + lax.axis_index("c")
    base = wid * _CHUNK
    pltpu.sync_copy(w_hbm.at[pl.ds(base, _CHUNK)], w_v)
    pltpu.sync_copy(z_hbm.at[pl.ds(base, _CHUNK)], z_v)

    @plsc.parallel_loop(0, _VECS, step=1, unroll=_UNROLL)
    def _step(i):
        off = i * _L
        x = (w_v[pl.ds(off, _L)] - z_v[pl.ds(off, _L)]) * _INV_T
        o_v[pl.ds(off, _L)] = 1.0 / (1.0 + jnp.exp(-x))
    pltpu.sync_copy(o_v, out_hbm.at[pl.ds(base, _CHUNK)])


def kernel(weights, noises):
    mesh = plsc.VectorSubcoreMesh(core_axis_name="c", subcore_axis_name="s")
    run = pl.kernel(
        _sc_body,
        mesh=mesh,
        out_type=jax.ShapeDtypeStruct((_N,), jnp.float32),
        scratch_types=[
            pltpu.VMEM((_CHUNK,), jnp.float32),
            pltpu.VMEM((_CHUNK,), jnp.float32),
            pltpu.VMEM((_CHUNK,), jnp.float32),
        ],
    )
    return run(weights, noises)


# speedup vs baseline: 2.3108x; 2.2883x over previous
import jax
import jax.numpy as jnp
from jax.experimental import pallas as pl

_N = 1024 * 1024
_INV_T = 10.0
_GRID = 16


def _body(w_ref, z_ref, o_ref):
    x = (w_ref[...] - z_ref[...]) * _INV_T
    o_ref[...] = jax.nn.sigmoid(x)


def kernel(weights, noises):
    blk = _N // _GRID
    out = pl.pallas_call(
        _body,
        grid=(_GRID,),
        in_specs=[
            pl.BlockSpec((blk,), lambda i: (i,)),
            pl.BlockSpec((blk,), lambda i: (i,)),
        ],
        out_specs=pl.BlockSpec((blk,), lambda i: (i,)),
        out_shape=jax.ShapeDtypeStruct((_N,), jnp.float32),
    )(weights, noises)
    return out


# speedup vs baseline: 4.3905x; 1.9000x over previous
import jax
import jax.numpy as jnp
from jax.experimental import pallas as pl

_N = 1024 * 1024
_INV_T = 10.0
_GRID = 4


def _body(w_ref, z_ref, o_ref):
    x = (w_ref[...] - z_ref[...]) * _INV_T
    o_ref[...] = jax.nn.sigmoid(x)


def kernel(weights, noises):
    blk = _N // _GRID
    out = pl.pallas_call(
        _body,
        grid=(_GRID,),
        in_specs=[
            pl.BlockSpec((blk,), lambda i: (i,)),
            pl.BlockSpec((blk,), lambda i: (i,)),
        ],
        out_specs=pl.BlockSpec((blk,), lambda i: (i,)),
        out_shape=jax.ShapeDtypeStruct((_N,), jnp.float32),
    )(weights, noises)
    return out


# speedup vs baseline: 5.1382x; 1.1703x over previous
import jax
import jax.numpy as jnp
from jax.experimental import pallas as pl

_N = 1024 * 1024
_INV_T = 10.0
_GRID = 2


def _body(w_ref, z_ref, o_ref):
    x = (w_ref[...] - z_ref[...]) * _INV_T
    o_ref[...] = jax.nn.sigmoid(x)


def kernel(weights, noises):
    blk = _N // _GRID
    out = pl.pallas_call(
        _body,
        grid=(_GRID,),
        in_specs=[
            pl.BlockSpec((blk,), lambda i: (i,)),
            pl.BlockSpec((blk,), lambda i: (i,)),
        ],
        out_specs=pl.BlockSpec((blk,), lambda i: (i,)),
        out_shape=jax.ShapeDtypeStruct((_N,), jnp.float32),
    )(weights, noises)
    return out
